# Initial kernel scaffold; baseline (speedup 1.0000x reference)
import jax
import jax.numpy as jnp
from jax.experimental import pallas as pl


def kernel(x, edge_index, W1, b1, W2, b2):
    raise NotImplementedError("write your pallas kernel here")


# speedup vs baseline: 11.7258x; 11.7258x over previous
import functools

import jax
import jax.numpy as jnp
from jax import lax
from jax.experimental import pallas as pl
from jax.experimental.pallas import tpu as pltpu
from jax.experimental.pallas import tpu_sc as plsc

NC = 2
NS = 16
NW = NC * NS
LANES = 16
K = 128


def _round_up(a, b):
    return (a + b - 1) // b * b


def _make_deg_kernel(e_pad, n_acc):
    ept = e_pad // NW
    rpt = n_acc // NS
    nchunks = ept // K
    mesh = plsc.VectorSubcoreMesh(
        core_axis_name="c", subcore_axis_name="s",
        num_cores=NC, num_subcores=NS)

    @functools.partial(
        pl.kernel,
        out_type=jax.ShapeDtypeStruct((NC, n_acc), jnp.float32),
        mesh=mesh,
        scratch_types=[
            pltpu.VMEM((K,), jnp.int32),
            pltpu.VMEM((K,), jnp.float32),
            pltpu.VMEM_SHARED((n_acc,), jnp.float32),
        ],
    )
    def deg_kernel(dst_hbm, zeros_hbm, out_hbm, idx_v, ones_v, acc_sh):
        c = lax.axis_index("c")
        s = lax.axis_index("s")
        pltpu.sync_copy(zeros_hbm.at[pl.ds(s * rpt, rpt)],
                        acc_sh.at[pl.ds(s * rpt, rpt)])
        for j in range(K // LANES):
            ones_v[pl.ds(j * LANES, LANES)] = jnp.ones((LANES,), jnp.float32)
        plsc.subcore_barrier()
        base = (c * NS + s) * ept

        def chunk(i, carry):
            pltpu.sync_copy(dst_hbm.at[pl.ds(base + i * K, K)], idx_v)
            pltpu.sync_copy(ones_v, acc_sh.at[idx_v], add=True)
            return carry

        lax.fori_loop(0, nchunks, chunk, 0)
        plsc.subcore_barrier()
        pltpu.sync_copy(acc_sh.at[pl.ds(s * rpt, rpt)],
                        out_hbm.at[c, pl.ds(s * rpt, rpt)])

    return deg_kernel


def _make_agg_kernel(e_pad, n_acc, d):
    ept = e_pad // NW
    rpt = n_acc // NS
    nchunks = ept // K
    mesh = plsc.VectorSubcoreMesh(
        core_axis_name="c", subcore_axis_name="s",
        num_cores=NC, num_subcores=NS)

    @functools.partial(
        pl.kernel,
        out_type=jax.ShapeDtypeStruct((NC, n_acc, d), jnp.float32),
        mesh=mesh,
        compiler_params=pltpu.CompilerParams(use_tc_tiling_on_sc=False),
        scratch_types=[
            pltpu.VMEM((2, K), jnp.int32),
            pltpu.VMEM((2, K), jnp.int32),
            pltpu.VMEM((2, K, d), jnp.float32),
            pltpu.SemaphoreType.DMA,
            pltpu.VMEM_SHARED((n_acc, d), jnp.float32),
        ],
    )
    def agg_kernel(src_hbm, dst_hbm, h_hbm, zeros_hbm, out_hbm,
                   sidx, didx, rows, sem, acc_sh):
        c = lax.axis_index("c")
        s = lax.axis_index("s")
        pltpu.sync_copy(zeros_hbm.at[pl.ds(s * rpt, rpt)],
                        acc_sh.at[pl.ds(s * rpt, rpt)])
        plsc.subcore_barrier()
        base = (c * NS + s) * ept

        def chunk(i, carry):
            pltpu.sync_copy(src_hbm.at[pl.ds(base + i * K, K)], sidx.at[0])
            pltpu.sync_copy(dst_hbm.at[pl.ds(base + i * K, K)], didx.at[0])
            pltpu.async_copy(h_hbm.at[sidx.at[0]], rows.at[0], sem).wait()
            pltpu.sync_copy(rows.at[0], acc_sh.at[didx.at[0]], add=True)
            return carry

        lax.fori_loop(0, nchunks, chunk, 0)
        plsc.subcore_barrier()
        pltpu.sync_copy(acc_sh.at[pl.ds(s * rpt, rpt)],
                        out_hbm.at[c, pl.ds(s * rpt, rpt)])

    return agg_kernel


_BLK = 1024


def _tc_scale_matmul(xp, degp, w1):
    n_acc, d_in = xp.shape
    d_h = w1.shape[1]

    def body(x_ref, deg_ref, w_ref, h_ref, dinv_ref):
        dinv = lax.rsqrt(deg_ref[0] + deg_ref[1] + 1.0)
        dinv_ref[...] = dinv
        h_ref[...] = jnp.dot(x_ref[...] * dinv, w_ref[...],
                             preferred_element_type=jnp.float32)

    return pl.pallas_call(
        body,
        grid=(n_acc // _BLK,),
        in_specs=[
            pl.BlockSpec((_BLK, d_in), lambda i: (i, 0)),
            pl.BlockSpec((NC, _BLK, 1), lambda i: (0, i, 0)),
            pl.BlockSpec((d_in, d_h), lambda i: (0, 0)),
        ],
        out_specs=[
            pl.BlockSpec((_BLK, d_h), lambda i: (i, 0)),
            pl.BlockSpec((_BLK, 1), lambda i: (i, 0)),
        ],
        out_shape=[
            jax.ShapeDtypeStruct((n_acc, d_h), jnp.float32),
            jax.ShapeDtypeStruct((n_acc, 1), jnp.float32),
        ],
    )(xp, degp, w1)


def _tc_mid(parts, h1p, dinv, b1, w2, n_real):
    _, n_acc, d_h = parts.shape
    d_o = w2.shape[1]

    def body(p_ref, h_ref, dinv_ref, b_ref, w_ref, o_ref):
        i = pl.program_id(0)
        dinv = dinv_ref[...]
        out1 = dinv * (p_ref[0] + p_ref[1] + h_ref[...]) + b_ref[...]
        h2 = dinv * jnp.dot(out1, w_ref[...],
                            preferred_element_type=jnp.float32)
        row = i * _BLK + lax.broadcasted_iota(jnp.int32, (_BLK, 1), 0)
        o_ref[...] = jnp.where(row < n_real, h2, 0.0)

    return pl.pallas_call(
        body,
        grid=(n_acc // _BLK,),
        in_specs=[
            pl.BlockSpec((NC, _BLK, d_h), lambda i: (0, i, 0)),
            pl.BlockSpec((_BLK, d_h), lambda i: (i, 0)),
            pl.BlockSpec((_BLK, 1), lambda i: (i, 0)),
            pl.BlockSpec((1, d_h), lambda i: (0, 0)),
            pl.BlockSpec((d_h, d_o), lambda i: (0, 0)),
        ],
        out_specs=pl.BlockSpec((_BLK, d_o), lambda i: (i, 0)),
        out_shape=jax.ShapeDtypeStruct((n_acc, d_o), jnp.float32),
    )(parts, h1p, dinv, b1, w2)


def _tc_final(parts, h2p, dinv, b2, n_real):
    _, n_acc, d_o = parts.shape
    blk = 1000 if n_real % 1000 == 0 else 8
    assert n_real % blk == 0

    def body(p_ref, h_ref, dinv_ref, b_ref, o_ref):
        t = dinv_ref[...] * (p_ref[0] + p_ref[1] + h_ref[...]) + b_ref[...]
        m = jnp.max(t, axis=1, keepdims=True)
        e = jnp.exp(t - m)
        o_ref[...] = e / jnp.sum(e, axis=1, keepdims=True)

    return pl.pallas_call(
        body,
        grid=(n_real // blk,),
        in_specs=[
            pl.BlockSpec((NC, blk, d_o), lambda i: (0, i, 0)),
            pl.BlockSpec((blk, d_o), lambda i: (i, 0)),
            pl.BlockSpec((blk, 1), lambda i: (i, 0)),
            pl.BlockSpec((1, d_o), lambda i: (0, 0)),
        ],
        out_specs=pl.BlockSpec((blk, d_o), lambda i: (i, 0)),
        out_shape=jax.ShapeDtypeStruct((n_real, d_o), jnp.float32),
    )(parts, h2p, dinv, b2)


def kernel(x, edge_index, W1, b1, W2, b2):
    n, d_in = x.shape
    e = edge_index.shape[1]
    d_h = W1.shape[1]
    d_o = W2.shape[1]

    e_pad = _round_up(e, NW * K)
    n_acc = _round_up(n + 1, _BLK)

    pad = jnp.full((e_pad - e,), n, dtype=edge_index.dtype)
    srcp = jnp.concatenate([edge_index[0], pad])
    dstp = jnp.concatenate([edge_index[1], pad])
    xp = jnp.pad(x, ((0, n_acc - n), (0, 0)))
    zeros_1d = jnp.zeros((n_acc,), jnp.float32)
    zeros_h = jnp.zeros((n_acc, d_h), jnp.float32)
    zeros_o = jnp.zeros((n_acc, d_o), jnp.float32)

    degp = _make_deg_kernel(e_pad, n_acc)(dstp, zeros_1d)
    h1p, dinv = _tc_scale_matmul(xp, degp.reshape(NC, n_acc, 1), W1)
    parts1 = _make_agg_kernel(e_pad, n_acc, d_h)(srcp, dstp, h1p, zeros_h)
    h2p = _tc_mid(parts1, h1p, dinv, b1.reshape(1, d_h), W2, n)
    parts2 = _make_agg_kernel(e_pad, n_acc, d_o)(srcp, dstp, h2p, zeros_o)
    return _tc_final(parts2, h2p, dinv, b2.reshape(1, d_o), n)


# speedup vs baseline: 12.1824x; 1.0389x over previous
import functools

import jax
import jax.numpy as jnp
from jax import lax
from jax.experimental import pallas as pl
from jax.experimental.pallas import tpu as pltpu
from jax.experimental.pallas import tpu_sc as plsc

NC = 2
NS = 16
NW = NC * NS
LANES = 16
K = 128
NSLOT = 8


def _round_up(a, b):
    return (a + b - 1) // b * b


def _make_deg_kernel(e_pad, n_acc):
    ept = e_pad // NW
    rpt = n_acc // NS
    nchunks = ept // K
    win = 8
    mesh = plsc.VectorSubcoreMesh(
        core_axis_name="c", subcore_axis_name="s",
        num_cores=NC, num_subcores=NS)

    @functools.partial(
        pl.kernel,
        out_type=jax.ShapeDtypeStruct((NC, n_acc), jnp.float32),
        mesh=mesh,
        scratch_types=[
            pltpu.VMEM((nchunks, K), jnp.int32),
            pltpu.VMEM((K,), jnp.float32),
            pltpu.SemaphoreType.DMA,
            pltpu.VMEM_SHARED((n_acc,), jnp.float32),
        ],
    )
    def deg_kernel(dst_hbm, zeros_hbm, out_hbm, idx_v, ones_v, sem, acc_sh):
        c = lax.axis_index("c")
        s = lax.axis_index("s")
        w = c * NS + s
        pltpu.sync_copy(zeros_hbm.at[pl.ds(s * rpt, rpt)],
                        acc_sh.at[pl.ds(s * rpt, rpt)])
        for j in range(K // LANES):
            ones_v[pl.ds(j * LANES, LANES)] = jnp.ones((LANES,), jnp.float32)
        pltpu.sync_copy(dst_hbm.at[w], idx_v)
        plsc.subcore_barrier()

        def chunk(i, carry):
            pltpu.async_copy(ones_v, acc_sh.at[idx_v.at[i]], sem, add=True)

            @pl.when(i >= win)
            def _():
                pltpu.make_async_copy(ones_v, acc_sh.at[idx_v.at[0]],
                                      sem).wait()
            return carry

        lax.fori_loop(0, nchunks, chunk, 0)
        for _ in range(min(win, nchunks)):
            pltpu.make_async_copy(ones_v, acc_sh.at[idx_v.at[0]], sem).wait()
        plsc.subcore_barrier()
        pltpu.sync_copy(acc_sh.at[pl.ds(s * rpt, rpt)],
                        out_hbm.at[c, pl.ds(s * rpt, rpt)])

    return deg_kernel


def _make_agg_kernel(e_pad, n_acc, d, n_pass):
    ept = e_pad // NW
    rpt = n_acc // NS
    nchunks = ept // K
    noct = nchunks // NSLOT
    mesh = plsc.VectorSubcoreMesh(
        core_axis_name="c", subcore_axis_name="s",
        num_cores=NC, num_subcores=NS)

    @functools.partial(
        pl.kernel,
        out_type=jax.ShapeDtypeStruct((n_pass, NC, n_acc, d), jnp.float32),
        mesh=mesh,
        compiler_params=pltpu.CompilerParams(use_tc_tiling_on_sc=False),
        scratch_types=[
            pltpu.VMEM((nchunks, K), jnp.int32),
            pltpu.VMEM((nchunks, K), jnp.int32),
            pltpu.VMEM((NSLOT, K, d), jnp.float32),
            pltpu.SemaphoreType.DMA((NSLOT,)),
            pltpu.SemaphoreType.DMA((NSLOT,)),
            pltpu.VMEM_SHARED((n_acc, d), jnp.float32),
        ],
    )
    def agg_kernel(src_hbm, dst_hbm, *rest):
        hs = rest[:n_pass]
        zeros_hbm, out_hbm, sidx, didx, rows, gsem, ssem, acc_sh = \
            rest[n_pass:]
        c = lax.axis_index("c")
        s = lax.axis_index("s")
        w = c * NS + s
        pltpu.sync_copy(src_hbm.at[w], sidx)
        pltpu.sync_copy(dst_hbm.at[w], didx)

        def wait_gather(b, h_hbm):
            pltpu.make_async_copy(h_hbm.at[sidx.at[0]], rows.at[b],
                                  gsem.at[b]).wait()

        def wait_scatter(b):
            pltpu.make_async_copy(rows.at[b], acc_sh.at[didx.at[0]],
                                  ssem.at[b]).wait()

        for p in range(n_pass):
            h_hbm = hs[p]
            pltpu.sync_copy(zeros_hbm.at[pl.ds(s * rpt, rpt)],
                            acc_sh.at[pl.ds(s * rpt, rpt)])
            plsc.subcore_barrier()

            def oct(q, carry):
                j0 = q * NSLOT
                for b in range(NSLOT):
                    j = j0 + b

                    @pl.when(q > 0)
                    def _(b=b):
                        wait_scatter(b)

                    pltpu.async_copy(h_hbm.at[sidx.at[j]], rows.at[b],
                                     gsem.at[b])
                    bb = (b + 4) % NSLOT
                    jj = j - 4

                    @pl.when(jj >= 0)
                    def _(bb=bb, jj=jj):
                        wait_gather(bb, h_hbm)
                        pltpu.async_copy(rows.at[bb],
                                         acc_sh.at[didx.at[jj]],
                                         ssem.at[bb], add=True)
                return carry

            lax.fori_loop(0, noct, oct, 0)
            for b in range(4):
                wait_gather(4 + b, h_hbm)
                pltpu.async_copy(rows.at[4 + b],
                                 acc_sh.at[didx.at[nchunks - 4 + b]],
                                 ssem.at[4 + b], add=True)
            for b in range(NSLOT):
                wait_scatter(b)
            plsc.subcore_barrier()
            pltpu.sync_copy(acc_sh.at[pl.ds(s * rpt, rpt)],
                            out_hbm.at[p, c, pl.ds(s * rpt, rpt)])
            if p + 1 < n_pass:
                plsc.subcore_barrier()

    return agg_kernel


_BLK = 1024


def _tc_scale_matmul(xp, degp, w1):
    n_acc, d_in = xp.shape
    d_h = w1.shape[1]
    dhalf = d_h // 2

    def body(x_ref, deg_ref, w_ref, ha_ref, hb_ref, dinv_ref):
        dinv = lax.rsqrt(deg_ref[0] + deg_ref[1] + 1.0)
        dinv_ref[...] = dinv
        h = jnp.dot(x_ref[...] * dinv, w_ref[...],
                    preferred_element_type=jnp.float32)
        ha_ref[...] = h[:, :dhalf]
        hb_ref[...] = h[:, dhalf:]

    return pl.pallas_call(
        body,
        grid=(n_acc // _BLK,),
        in_specs=[
            pl.BlockSpec((_BLK, d_in), lambda i: (i, 0)),
            pl.BlockSpec((NC, _BLK, 1), lambda i: (0, i, 0)),
            pl.BlockSpec((d_in, d_h), lambda i: (0, 0)),
        ],
        out_specs=[
            pl.BlockSpec((_BLK, dhalf), lambda i: (i, 0)),
            pl.BlockSpec((_BLK, dhalf), lambda i: (i, 0)),
            pl.BlockSpec((_BLK, 1), lambda i: (i, 0)),
        ],
        out_shape=[
            jax.ShapeDtypeStruct((n_acc, dhalf), jnp.float32),
            jax.ShapeDtypeStruct((n_acc, dhalf), jnp.float32),
            jax.ShapeDtypeStruct((n_acc, 1), jnp.float32),
        ],
    )(xp, degp, w1)


def _tc_mid(parts, h1a, h1b, dinv, b1, w2, n_real):
    n_pass, _, n_acc, dhalf = parts.shape
    d_h = 2 * dhalf
    d_o = w2.shape[1]

    def body(p_ref, ha_ref, hb_ref, dinv_ref, b_ref, w_ref, o_ref):
        i = pl.program_id(0)
        dinv = dinv_ref[...]
        agg = jnp.concatenate(
            [p_ref[0, 0] + p_ref[0, 1] + ha_ref[...],
             p_ref[1, 0] + p_ref[1, 1] + hb_ref[...]], axis=1)
        out1 = dinv * agg + b_ref[...]
        h2 = dinv * jnp.dot(out1, w_ref[...],
                            preferred_element_type=jnp.float32)
        row = i * _BLK + lax.broadcasted_iota(jnp.int32, (_BLK, 1), 0)
        o_ref[...] = jnp.where(row < n_real, h2, 0.0)

    return pl.pallas_call(
        body,
        grid=(n_acc // _BLK,),
        in_specs=[
            pl.BlockSpec((n_pass, NC, _BLK, dhalf), lambda i: (0, 0, i, 0)),
            pl.BlockSpec((_BLK, dhalf), lambda i: (i, 0)),
            pl.BlockSpec((_BLK, dhalf), lambda i: (i, 0)),
            pl.BlockSpec((_BLK, 1), lambda i: (i, 0)),
            pl.BlockSpec((1, d_h), lambda i: (0, 0)),
            pl.BlockSpec((d_h, d_o), lambda i: (0, 0)),
        ],
        out_specs=pl.BlockSpec((_BLK, d_o), lambda i: (i, 0)),
        out_shape=jax.ShapeDtypeStruct((n_acc, d_o), jnp.float32),
    )(parts, h1a, h1b, dinv, b1, w2)


def _tc_final(parts, h2p, dinv, b2, n_real):
    _, n_acc, d_o = parts.shape
    blk = 1000 if n_real % 1000 == 0 else 8
    assert n_real % blk == 0

    def body(p_ref, h_ref, dinv_ref, b_ref, o_ref):
        t = dinv_ref[...] * (p_ref[0] + p_ref[1] + h_ref[...]) + b_ref[...]
        m = jnp.max(t, axis=1, keepdims=True)
        e = jnp.exp(t - m)
        o_ref[...] = e / jnp.sum(e, axis=1, keepdims=True)

    return pl.pallas_call(
        body,
        grid=(n_real // blk,),
        in_specs=[
            pl.BlockSpec((NC, blk, d_o), lambda i: (0, i, 0)),
            pl.BlockSpec((blk, d_o), lambda i: (i, 0)),
            pl.BlockSpec((blk, 1), lambda i: (i, 0)),
            pl.BlockSpec((1, d_o), lambda i: (0, 0)),
        ],
        out_specs=pl.BlockSpec((blk, d_o), lambda i: (i, 0)),
        out_shape=jax.ShapeDtypeStruct((n_real, d_o), jnp.float32),
    )(parts, h2p, dinv, b2)


def kernel(x, edge_index, W1, b1, W2, b2):
    n, d_in = x.shape
    e = edge_index.shape[1]
    d_h = W1.shape[1]
    d_o = W2.shape[1]
    dhalf = d_h // 2

    e_pad = _round_up(e, NW * K * NSLOT)
    ept = e_pad // NW
    n_acc = _round_up(n + 1, _BLK)

    pad = jnp.full((e_pad - e,), n, dtype=edge_index.dtype)
    srcp = jnp.concatenate([edge_index[0], pad])
    dstp = jnp.concatenate([edge_index[1], pad])
    src3 = srcp.reshape(NW, ept // K, K)
    dst3 = dstp.reshape(NW, ept // K, K)
    xp = jnp.pad(x, ((0, n_acc - n), (0, 0)))
    zeros_1d = jnp.zeros((n_acc,), jnp.float32)
    zeros_h = jnp.zeros((n_acc, dhalf), jnp.float32)
    zeros_o = jnp.zeros((n_acc, d_o), jnp.float32)

    degp = _make_deg_kernel(e_pad, n_acc)(dst3, zeros_1d)
    h1a, h1b, dinv = _tc_scale_matmul(xp, degp.reshape(NC, n_acc, 1), W1)
    parts1 = _make_agg_kernel(e_pad, n_acc, dhalf, 2)(
        src3, dst3, h1a, h1b, zeros_h)
    h2p = _tc_mid(parts1, h1a, h1b, dinv, b1.reshape(1, d_h), W2, n)
    parts2 = _make_agg_kernel(e_pad, n_acc, d_o, 1)(
        src3, dst3, h2p, zeros_o)
    return _tc_final(parts2.reshape(NC, n_acc, d_o), h2p, dinv,
                     b2.reshape(1, d_o), n)
